# baseline (speedup 1.0000x reference)
import functools

import jax
import jax.numpy as jnp
from jax import lax
from jax.experimental import pallas as pl
from jax.experimental.pallas import tpu as pltpu
from jax.experimental.pallas import tpu_sc as plsc

D_MODEL = 128
N_TOKENS = 1024 * 200

NUM_CORES = 2
NUM_SUBCORES = 16
NUM_WORKERS = NUM_CORES * NUM_SUBCORES

B_PER_W = N_TOKENS // NUM_WORKERS
CHUNK = 128
N_CHUNKS = B_PER_W // CHUNK
NBUF = 5
N_GROUPS = N_CHUNKS // NBUF


def _body(tok_hbm, emb_hbm, out_hbm, idx_v, bufs, gsems, osems):
    wid = lax.axis_index("s") * NUM_CORES + lax.axis_index("c")
    base = wid * B_PER_W

    pltpu.sync_copy(tok_hbm.at[wid], idx_v)

    def gather_start(j, b):
        pltpu.async_copy(emb_hbm.at[idx_v.at[j]], bufs[b], gsems[b])

    def scatter_wait(j, b):
        pltpu.make_async_copy(
            bufs[b], out_hbm.at[pl.ds(base + j * CHUNK, CHUNK)], osems[b]
        ).wait()

    for b in range(NBUF):
        gather_start(b, b)

    DEFER = 2

    def group(g, _):
        for b in range(NBUF):
            i = g * NBUF + b
            pltpu.make_async_copy(emb_hbm.at[idx_v.at[i]], bufs[b], gsems[b]).wait()
            pltpu.async_copy(
                bufs[b], out_hbm.at[pl.ds(base + i * CHUNK, CHUNK)], osems[b]
            )

            b2 = (b - DEFER) % NBUF

            @pl.when((i >= DEFER) & (i - DEFER + NBUF < N_CHUNKS))
            def _():
                scatter_wait(i - DEFER, b2)
                gather_start(i - DEFER + NBUF, b2)

        return 0

    lax.fori_loop(0, N_GROUPS, group, 0)

    for j in range(N_CHUNKS - NBUF, N_CHUNKS):
        scatter_wait(j, j % NBUF)


@jax.jit
def _embed(tokens_flat, emb):
    mesh = plsc.VectorSubcoreMesh(core_axis_name="c", subcore_axis_name="s")
    tok3 = tokens_flat.reshape(NUM_WORKERS, N_CHUNKS, CHUNK)
    run = pl.kernel(
        _body,
        out_type=jax.ShapeDtypeStruct((N_TOKENS, D_MODEL), jnp.float32),
        mesh=mesh,
        scratch_types=[
            pltpu.VMEM((N_CHUNKS, CHUNK), jnp.int32),
            [pltpu.VMEM((CHUNK, D_MODEL), jnp.float32) for _ in range(NBUF)],
            [pltpu.SemaphoreType.DMA for _ in range(NBUF)],
            [pltpu.SemaphoreType.DMA for _ in range(NBUF)],
        ],
    )
    return run(tok3, emb)


def kernel(tokens, emb):
    tokens_flat = tokens.reshape(-1).astype(jnp.int32)
    out = _embed(tokens_flat, emb)
    return out.reshape(tokens.shape + (D_MODEL,))
